# baseline (speedup 1.0000x reference)
import jax
import jax.numpy as jnp
from jax import lax
from jax.experimental import pallas as pl
from jax.experimental.pallas import tpu as pltpu
from jax.experimental.pallas import tpu_sc as plsc

B = 4096
N = 8192
C = 1000
L = 16
CP = 1008
NC = 2
NS = 16
NW = NC * NS
ROWS_PER_TILE = B // NW

QI = 4
NQ = ROWS_PER_TILE // QI
GO = 16
GW = GO * C
OB = 16128
N_OG = ROWS_PER_TILE // GO
OUTER = N_OG // 2

SCAT_U = 32
AMAX_U = 9


def _body(in_hbm, out_hbm, in0, in1, ob0, ob1, hist_v, bases_sm,
          si0, si1, so0, so1):
  wid = lax.axis_index("s") * NC + lax.axis_index("c")
  row0 = wid * ROWS_PER_TILE

  inbufs = (in0, in1)
  sems_in = (si0, si1)
  outbufs = (ob0, ob1)
  sems_out = (so0, so1)

  zf = jnp.zeros((L,), jnp.float32)
  zi = jnp.zeros((L,), jnp.int32)
  ones = jnp.ones((L,), jnp.int32)
  lanes = lax.iota(jnp.int32, L)
  rinit = (CP - 1) - lanes

  def zero_hist0(i, _):
    for u in range(AMAX_U):
      hist_v[pl.ds((i * AMAX_U + u) * L, L)] = zi
    return 0
  lax.fori_loop(0, CP // (L * AMAX_U), zero_hist0, 0)

  for s in range(2):
    pltpu.async_copy(in_hbm.at[pl.ds((row0 + s * QI) * N, QI * N)],
                     inbufs[s], sems_in[s])

  def process_row(buf, roff, out_ref, r8, slot):
    def zero_hist(i, _):
      for u in range(AMAX_U):
        hist_v[pl.ds((i * AMAX_U + u) * L, L)] = zi
      return 0
    lax.fori_loop(0, CP // (L * AMAX_U), zero_hist, 0)

    def scatter(i, _):
      idxs = [buf[pl.ds(roff + (i * SCAT_U + u) * L, L)]
              for u in range(SCAT_U)]
      for idx in idxs:
        plsc.addupdate_scatter(hist_v, [idx], ones)
      return 0
    lax.fori_loop(0, N // (L * SCAT_U), scatter, 0)

    def amax(i, carry):
      bk, r = carry
      for u in range(AMAX_U):
        c = hist_v[pl.ds((i * AMAX_U + u) * L, L)]
        bk = jnp.maximum(bk, (c << 11) + r)
        r = r - L
      return bk, r
    bk, _ = lax.fori_loop(0, CP // (L * AMAX_U), amax,
                          (jnp.full((L,), -1, jnp.int32), rinit))
    mode = (CP - 1) - (jnp.max(bk) & 2047)

    base = (mode // L) * L
    vec = (lanes + base == mode).astype(jnp.float32)
    off = r8 * C + base
    out_ref[pl.ds(off, L)] = vec
    bases_sm[slot * GO + r8] = off

  def outer(g2, _):
    for o_local in range(2):
      og = g2 * 2 + o_local
      ob = outbufs[o_local]
      @pl.when(g2 > 0)
      def _wait_and_clear():
        pltpu.make_async_copy(ob.at[pl.ds(0, GW)], out_hbm.at[pl.ds(0, GW)],
                              sems_out[o_local]).wait()
        for r8 in range(GO):
          ob[pl.ds(bases_sm[o_local * GO + r8], L)] = zf

      @pl.when(g2 == 0)
      def _zero_full():
        def zero_out(i, _):
          for u in range(8):
            ob[pl.ds((i * 8 + u) * L, L)] = zf
          return 0
        lax.fori_loop(0, OB // (L * 8), zero_out, 0)

      def jbody(j, _):
        for q in range(2):
          qq = j * 2 + q
          gq = og * (GO // QI) + qq
          pltpu.make_async_copy(in_hbm.at[pl.ds(0, QI * N)], inbufs[q],
                                sems_in[q]).wait()
          for rr in range(QI):
            process_row(inbufs[q], rr * N, ob, qq * QI + rr, o_local)
          @pl.when(gq + 2 < NQ)
          def _prefetch():
            pltpu.async_copy(
                in_hbm.at[pl.ds((row0 + (gq + 2) * QI) * N, QI * N)],
                inbufs[q], sems_in[q])
        return 0
      lax.fori_loop(0, GO // QI // 2, jbody, 0)

      pltpu.async_copy(ob.at[pl.ds(0, GW)],
                       out_hbm.at[pl.ds((row0 + og * GO) * C, GW)],
                       sems_out[o_local])
    return 0

  lax.fori_loop(0, OUTER, outer, 0)

  for o_local in range(2):
    pltpu.make_async_copy(outbufs[o_local].at[pl.ds(0, GW)],
                          out_hbm.at[pl.ds(0, GW)],
                          sems_out[o_local]).wait()


@jax.jit
def kernel(inputs):
  mesh = plsc.VectorSubcoreMesh(
      core_axis_name="c", subcore_axis_name="s",
      num_cores=NC, num_subcores=NS)
  run = pl.kernel(
      _body,
      out_type=jax.ShapeDtypeStruct((B * C,), jnp.float32),
      mesh=mesh,
      scratch_types=(
          [pltpu.VMEM((QI * N,), jnp.int32) for _ in range(2)]
          + [pltpu.VMEM((OB,), jnp.float32) for _ in range(2)]
          + [pltpu.VMEM((CP,), jnp.int32)]
          + [pltpu.SMEM((2 * GO,), jnp.int32)]
          + [pltpu.SemaphoreType.DMA] * 4
      ),
      compiler_params=pltpu.CompilerParams(needs_layout_passes=False),
  )
  return run(inputs.reshape(B * N)).reshape(B, C)


# speedup vs baseline: 1.6747x; 1.6747x over previous
import jax
import jax.numpy as jnp
from jax import lax
from jax.experimental import pallas as pl
from jax.experimental.pallas import tpu as pltpu
from jax.experimental.pallas import tpu_sc as plsc

B = 4096
N = 8192
C = 1000
L = 16
CP = 1008
NC = 2
NS = 16
NW = NC * NS
ROWS_PER_TILE = B // NW

NBUF = 4
GO = 16
GW = GO * C
OB = 16128
N_OG = ROWS_PER_TILE // GO
OUTER = N_OG // 2

SCAT_U = 32
AMAX_U = 9


def _body(in_hbm, out_hbm, in0, in1, in2, in3, ob0, ob1, hist_v, bases_sm,
          si0, si1, si2, si3, so0, so1):
  wid = lax.axis_index("s") * NC + lax.axis_index("c")
  row0 = wid * ROWS_PER_TILE

  inbufs = (in0, in1, in2, in3)
  sems_in = (si0, si1, si2, si3)
  outbufs = (ob0, ob1)
  sems_out = (so0, so1)

  zf = jnp.zeros((L,), jnp.float32)
  zi = jnp.zeros((L,), jnp.int32)
  ones = jnp.ones((L,), jnp.int32)
  lanes = lax.iota(jnp.int32, L)
  rinit = (CP - 1) - lanes

  def zero_hist0(i, _):
    for u in range(AMAX_U):
      hist_v[pl.ds((i * AMAX_U + u) * L, L)] = zi
    return 0
  lax.fori_loop(0, CP // (L * AMAX_U), zero_hist0, 0)

  for s in range(NBUF):
    pltpu.async_copy(in_hbm.at[row0 + s], inbufs[s], sems_in[s])

  def process_row(buf, roff, out_ref, r8, slot):
    def zero_hist(i, _):
      for u in range(AMAX_U):
        hist_v[pl.ds((i * AMAX_U + u) * L, L)] = zi
      return 0
    lax.fori_loop(0, CP // (L * AMAX_U), zero_hist, 0)

    def scatter(i, _):
      idxs = [buf[pl.ds(roff + (i * SCAT_U + u) * L, L)]
              for u in range(SCAT_U)]
      for idx in idxs:
        plsc.addupdate_scatter(hist_v, [idx], ones)
      return 0
    lax.fori_loop(0, N // (L * SCAT_U), scatter, 0)

    def amax(i, carry):
      bk, r = carry
      for u in range(AMAX_U):
        c = hist_v[pl.ds((i * AMAX_U + u) * L, L)]
        bk = jnp.maximum(bk, (c << 11) + r)
        r = r - L
      return bk, r
    bk, _ = lax.fori_loop(0, CP // (L * AMAX_U), amax,
                          (jnp.full((L,), -1, jnp.int32), rinit))
    mode = (CP - 1) - (jnp.max(bk) & 2047)

    base = (mode // L) * L
    vec = (lanes + base == mode).astype(jnp.float32)
    off = r8 * C + base
    out_ref[pl.ds(off, L)] = vec
    bases_sm[slot * GO + r8] = off

  def outer(g2, _):
    for o_local in range(2):
      og = g2 * 2 + o_local
      ob = outbufs[o_local]
      @pl.when(g2 > 0)
      def _wait_and_clear():
        pltpu.make_async_copy(ob.at[pl.ds(0, GW)], out_hbm.at[pl.ds(0, GW)],
                              sems_out[o_local]).wait()
        for r8 in range(GO):
          ob[pl.ds(bases_sm[o_local * GO + r8], L)] = zf

      @pl.when(g2 == 0)
      def _zero_full():
        def zero_out(i, _):
          for u in range(8):
            ob[pl.ds((i * 8 + u) * L, L)] = zf
          return 0
        lax.fori_loop(0, OB // (L * 8), zero_out, 0)

      def quad(j, _):
        for rr in range(NBUF):
          r8 = j * NBUF + rr
          row = og * GO + r8
          pltpu.make_async_copy(in_hbm.at[0], inbufs[rr],
                                sems_in[rr]).wait()
          process_row(inbufs[rr], 0, ob, r8, o_local)
          @pl.when(row + NBUF < ROWS_PER_TILE)
          def _prefetch():
            pltpu.async_copy(in_hbm.at[row0 + row + NBUF], inbufs[rr],
                             sems_in[rr])
        return 0
      lax.fori_loop(0, GO // NBUF, quad, 0)

      pltpu.async_copy(ob.at[pl.ds(0, GW)],
                       out_hbm.at[pl.ds((row0 + og * GO) * C, GW)],
                       sems_out[o_local])
    return 0

  lax.fori_loop(0, OUTER, outer, 0)

  for o_local in range(2):
    pltpu.make_async_copy(outbufs[o_local].at[pl.ds(0, GW)],
                          out_hbm.at[pl.ds(0, GW)],
                          sems_out[o_local]).wait()


@jax.jit
def kernel(inputs):
  mesh = plsc.VectorSubcoreMesh(
      core_axis_name="c", subcore_axis_name="s",
      num_cores=NC, num_subcores=NS)
  run = pl.kernel(
      _body,
      out_type=jax.ShapeDtypeStruct((B * C,), jnp.float32),
      mesh=mesh,
      scratch_types=(
          [pltpu.VMEM((N,), jnp.int32) for _ in range(NBUF)]
          + [pltpu.VMEM((OB,), jnp.float32) for _ in range(2)]
          + [pltpu.VMEM((CP,), jnp.int32)]
          + [pltpu.SMEM((2 * GO,), jnp.int32)]
          + [pltpu.SemaphoreType.DMA] * 6
      ),
      compiler_params=pltpu.CompilerParams(needs_layout_passes=False),
  )
  return run(inputs).reshape(B, C)


# speedup vs baseline: 1.7434x; 1.0410x over previous
import jax
import jax.numpy as jnp
from jax import lax
from jax.experimental import pallas as pl
from jax.experimental.pallas import tpu as pltpu
from jax.experimental.pallas import tpu_sc as plsc

B = 4096
N = 8192
C = 1000
L = 16
CP = 1008
NC = 2
NS = 16
NW = NC * NS
ROWS_PER_TILE = B // NW

NBUF = 4
GO = 16
GW = GO * C
OB = 16128
N_OG = ROWS_PER_TILE // GO
OUTER = N_OG // 2

SCAT_U = 32
AMAX_U = 9


def _body(in_hbm, out_hbm, in0, in1, in2, in3, ob0, ob1, hist_v, bases_sm,
          si0, si1, si2, si3, so0, so1):
  wid = lax.axis_index("s") * NC + lax.axis_index("c")
  row0 = wid * ROWS_PER_TILE

  inbufs = (in0, in1, in2, in3)
  sems_in = (si0, si1, si2, si3)
  outbufs = (ob0, ob1)
  sems_out = (so0, so1)

  zf = jnp.zeros((L,), jnp.float32)
  zi = jnp.zeros((L,), jnp.int32)
  ones = jnp.ones((L,), jnp.int32)
  lanes = lax.iota(jnp.int32, L)
  rinit = (CP - 1) - lanes

  def zero_hist0(i, _):
    for u in range(AMAX_U):
      hist_v[pl.ds((i * AMAX_U + u) * L, L)] = zi
    return 0
  lax.fori_loop(0, CP // (L * AMAX_U), zero_hist0, 0)

  for s in range(NBUF):
    pltpu.async_copy(in_hbm.at[row0 + s], inbufs[s], sems_in[s])

  def process_row(buf, roff, out_ref, r8, slot):
    def scatter(i, _):
      idxs = [buf[pl.ds(roff + (i * SCAT_U + u) * L, L)]
              for u in range(SCAT_U)]
      for idx in idxs:
        plsc.addupdate_scatter(hist_v, [idx], ones)
      return 0
    lax.fori_loop(0, N // (L * SCAT_U), scatter, 0)

    def amax(i, carry):
      bk, r = carry
      for u in range(AMAX_U):
        c = hist_v[pl.ds((i * AMAX_U + u) * L, L)]
        bk = jnp.maximum(bk, (c << 11) + r)
        r = r - L
      @pl.when(i > 0)
      def _zero_prev():
        for u in range(AMAX_U):
          hist_v[pl.ds(((i - 1) * AMAX_U + u) * L, L)] = zi
      return bk, r
    bk, _ = lax.fori_loop(0, CP // (L * AMAX_U), amax,
                          (jnp.full((L,), -1, jnp.int32), rinit))
    for u in range(AMAX_U):
      hist_v[pl.ds((CP // (L * AMAX_U) - 1) * AMAX_U * L + u * L, L)] = zi
    mode = (CP - 1) - (jnp.max(bk) & 2047)

    base = (mode // L) * L
    vec = (lanes + base == mode).astype(jnp.float32)
    off = r8 * C + base
    out_ref[pl.ds(off, L)] = vec
    bases_sm[slot * GO + r8] = off

  def outer(g2, _):
    for o_local in range(2):
      og = g2 * 2 + o_local
      ob = outbufs[o_local]
      @pl.when(g2 > 0)
      def _wait_and_clear():
        pltpu.make_async_copy(ob.at[pl.ds(0, GW)], out_hbm.at[pl.ds(0, GW)],
                              sems_out[o_local]).wait()
        for r8 in range(GO):
          ob[pl.ds(bases_sm[o_local * GO + r8], L)] = zf

      @pl.when(g2 == 0)
      def _zero_full():
        def zero_out(i, _):
          for u in range(8):
            ob[pl.ds((i * 8 + u) * L, L)] = zf
          return 0
        lax.fori_loop(0, OB // (L * 8), zero_out, 0)

      def quad(j, _):
        for rr in range(NBUF):
          r8 = j * NBUF + rr
          row = og * GO + r8
          pltpu.make_async_copy(in_hbm.at[0], inbufs[rr],
                                sems_in[rr]).wait()
          process_row(inbufs[rr], 0, ob, r8, o_local)
          @pl.when(row + NBUF < ROWS_PER_TILE)
          def _prefetch():
            pltpu.async_copy(in_hbm.at[row0 + row + NBUF], inbufs[rr],
                             sems_in[rr])
        return 0
      lax.fori_loop(0, GO // NBUF, quad, 0)

      pltpu.async_copy(ob.at[pl.ds(0, GW)],
                       out_hbm.at[pl.ds((row0 + og * GO) * C, GW)],
                       sems_out[o_local])
    return 0

  lax.fori_loop(0, OUTER, outer, 0)

  for o_local in range(2):
    pltpu.make_async_copy(outbufs[o_local].at[pl.ds(0, GW)],
                          out_hbm.at[pl.ds(0, GW)],
                          sems_out[o_local]).wait()


@jax.jit
def kernel(inputs):
  mesh = plsc.VectorSubcoreMesh(
      core_axis_name="c", subcore_axis_name="s",
      num_cores=NC, num_subcores=NS)
  run = pl.kernel(
      _body,
      out_type=jax.ShapeDtypeStruct((B * C,), jnp.float32),
      mesh=mesh,
      scratch_types=(
          [pltpu.VMEM((N,), jnp.int32) for _ in range(NBUF)]
          + [pltpu.VMEM((OB,), jnp.float32) for _ in range(2)]
          + [pltpu.VMEM((CP,), jnp.int32)]
          + [pltpu.SMEM((2 * GO,), jnp.int32)]
          + [pltpu.SemaphoreType.DMA] * 6
      ),
      compiler_params=pltpu.CompilerParams(needs_layout_passes=False),
  )
  return run(inputs).reshape(B, C)
